# baseline (speedup 1.0000x reference)
import functools

import jax
import jax.numpy as jnp
from jax import lax
from jax.experimental import pallas as pl
from jax.experimental.pallas import tpu as pltpu
from jax.experimental.pallas import tpu_sc as plsc

B = 16384
L = 50
N = B * L
D = 32
V = 1000000
NC = 2
NS = 16
NW = NC * NS
PER_W = N // NW
CHUNK = 128
NCHUNK = PER_W // CHUNK
NBUF = 4
GRP = CHUNK // 16

TCOLS = V // 128
TPW = TCOLS // NW
TREM = TCOLS - TPW * NW
TAIL = V - TCOLS * 128


def _transpose_body(embt_hbm, tail_hbm, out_hbm,
                    in_0, in_1, ob_0, ob_1, si_0, si_1, so_0, so_1):
    in_v = (in_0, in_1)
    out_v = (ob_0, ob_1)
    sem_in = (si_0, si_1)
    sem_out = (so_0, so_1)

    wid = lax.axis_index("s") * NC + lax.axis_index("c")
    iota16 = lax.broadcasted_iota(jnp.int32, (16,), 0)
    d_lo = iota16
    d_hi = iota16 + 16

    def fire_in(j, b):
        tc = wid + NW * j
        for d in range(32):
            pltpu.async_copy(
                embt_hbm.at[d, pl.ds(tc * 128, 128)],
                in_v[b].at[d, pl.ds(0, 128)], sem_in[b])

    def wait_in(j, b):
        tc = wid + NW * j
        for d in range(32):
            pltpu.make_async_copy(
                embt_hbm.at[d, pl.ds(tc * 128, 128)],
                in_v[b].at[d, pl.ds(0, 128)],
                sem_in[b]).wait()

    def fire_out(j, b):
        tc = wid + NW * j
        pltpu.async_copy(out_v[b],
                         out_hbm.at[pl.ds(tc * 4096, 4096)], sem_out[b])

    def wait_out(j, b):
        tc = wid + NW * j
        pltpu.make_async_copy(out_v[b],
                              out_hbm.at[pl.ds(tc * 4096, 4096)],
                              sem_out[b]).wait()

    def compute(b):
        def grp(g, _):
            for i2 in range(16):
                col = jnp.full((16,), g * 16 + i2, jnp.int32)
                va = plsc.load_gather(in_v[b], [d_lo, col])
                vb = plsc.load_gather(in_v[b], [d_hi, col])
                row = plsc.pack(va, vb, format=plsc.PackFormat.INTERLEAVED)
                out_v[b][pl.ds((g * 16 + i2) * 32, 32)] = row
            return 0

        lax.fori_loop(0, 8, grp, 0)

    fire_in(0, 0)
    fire_in(1, 1)

    def body_j2(j2, _):
        for b in range(2):
            j = 2 * j2 + b
            wait_in(j, b)

            @pl.when(j2 > 0)
            def _():
                wait_out(j - 2, b)

            compute(b)
            fire_out(j, b)

            @pl.when(j + 2 < TPW)
            def _():
                fire_in(j + 2, b)
        return 0

    lax.fori_loop(0, TPW // 2, body_j2, 0)
    wait_out(TPW - 2, 0)
    wait_out(TPW - 1, 1)

    @pl.when(wid < TREM)
    def _():
        fire_in(TPW, 0)
        wait_in(TPW, 0)
        compute(0)
        fire_out(TPW, 0)
        wait_out(TPW, 0)

    @pl.when(wid == NW - 1)
    def _():
        pltpu.sync_copy(tail_hbm, out_v[1].at[pl.ds(0, TAIL * D)])
        pltpu.sync_copy(out_v[1].at[pl.ds(0, TAIL * D)],
                        out_hbm.at[pl.ds(TCOLS * 4096, TAIL * D)])


def _sc_body(idx1_hbm, idx2_hbm, emb_hbm, out_hbm,
             idx1_v, idx2_v, out_v,
             r1_0, r1_1, r1_2, r1_3,
             r2_0, r2_1, r2_2, r2_3,
             s0, s1, s2, s3):
    r1 = (r1_0, r1_1, r1_2, r1_3)
    r2 = (r2_0, r2_1, r2_2, r2_3)
    sems = (s0, s1, s2, s3)

    wid = lax.axis_index("s") * NC + lax.axis_index("c")

    pltpu.sync_copy(idx1_hbm.at[wid], idx1_v)
    pltpu.sync_copy(idx2_hbm.at[wid], idx2_v)

    def fire(c, b):
        pltpu.async_copy(emb_hbm.at[idx1_v.at[c]], r1[b], sems[b])
        pltpu.async_copy(emb_hbm.at[idx2_v.at[c]], r2[b], sems[b])

    for b in range(NBUF):
        fire(b, b)

    iota16 = lax.broadcasted_iota(jnp.int32, (16,), 0)
    rot_idx = {k: (iota16 + k) % 16 for k in (8, 4, 2, 1)}
    lane_masks = [iota16 == i2 for i2 in range(16)]

    def body_cg(cg, _):
        for b in range(NBUF):
            c = cg * NBUF + b
            pltpu.make_async_copy(emb_hbm.at[idx1_v.at[c]], r1[b], sems[b]).wait()
            pltpu.make_async_copy(emb_hbm.at[idx2_v.at[c]], r2[b], sems[b]).wait()

            def dot_g(g, _):
                merged = jnp.zeros((16,), jnp.float32)
                for i2 in range(16):
                    i = g * 16 + i2
                    a0, a1 = plsc.unpack(r1[b][i, :],
                                         format=plsc.PackFormat.INTERLEAVED)
                    b0, b1 = plsc.unpack(r2[b][i, :],
                                         format=plsc.PackFormat.INTERLEAVED)
                    p = a0 * b0 + a1 * b1
                    for k in (8, 4, 2, 1):
                        p = p + jnp.take(p, rot_idx[k])
                    merged = jnp.where(lane_masks[i2], p, merged)
                out_v[pl.ds(c * CHUNK + g * 16, 16)] = merged
                return 0

            lax.fori_loop(0, GRP, dot_g, 0)

            nxt = c + NBUF

            @pl.when(nxt < NCHUNK)
            def _():
                fire(nxt, b)
        return 0

    lax.fori_loop(0, NCHUNK // NBUF, body_cg, 0)

    pltpu.sync_copy(out_v, out_hbm.at[pl.ds(wid * PER_W, PER_W)])


@jax.jit
def kernel(inputs, embeddings):
    idx = inputs.astype(jnp.int32).reshape(2, NW, NCHUNK, CHUNK)
    mesh = plsc.VectorSubcoreMesh(core_axis_name="c", subcore_axis_name="s")

    embt = embeddings.T
    tail_bf = embeddings[TCOLS * 128:].astype(jnp.bfloat16).reshape(-1)
    k1 = functools.partial(
        pl.kernel,
        out_type=jax.ShapeDtypeStruct((V * D,), jnp.bfloat16),
        mesh=mesh,
        scratch_types=(
            [pltpu.VMEM((32, 133), jnp.float32)] * 2
            + [pltpu.VMEM((128 * D,), jnp.bfloat16)] * 2
            + [pltpu.SemaphoreType.DMA] * 4
        ),
        compiler_params=pltpu.CompilerParams(
            needs_layout_passes=False, use_tc_tiling_on_sc=False),
    )(_transpose_body)
    emb_bf = k1(embt, tail_bf).reshape(V, D)

    scratch = (
        [pltpu.VMEM((NCHUNK, CHUNK), jnp.int32)] * 2
        + [pltpu.VMEM((PER_W,), jnp.float32)]
        + [pltpu.VMEM((CHUNK, D), jnp.bfloat16)] * (2 * NBUF)
        + [pltpu.SemaphoreType.DMA] * NBUF
    )
    k2 = functools.partial(
        pl.kernel,
        out_type=jax.ShapeDtypeStruct((N,), jnp.float32),
        mesh=mesh,
        scratch_types=scratch,
        compiler_params=pltpu.CompilerParams(
            needs_layout_passes=False, use_tc_tiling_on_sc=False),
    )(_sc_body)
    out = k2(idx[0], idx[1], emb_bf)
    return out.reshape(B, L)


# speedup vs baseline: 5.0015x; 5.0015x over previous
import functools

import jax
import jax.numpy as jnp
from jax import lax
from jax.experimental import pallas as pl
from jax.experimental.pallas import tpu as pltpu
from jax.experimental.pallas import tpu_sc as plsc

B = 16384
L = 50
N = B * L
D = 32
NC = 2
NS = 16
NW = NC * NS
PER_W = N // NW
CHUNK = 128
NCHUNK = PER_W // CHUNK
NBUF = 4
GRP = CHUNK // 16


def _sc_body(idx1_hbm, idx2_hbm, emb_hbm, out_hbm,
             idx1_v, idx2_v, out_v,
             r1_0, r1_1, r1_2, r1_3,
             r2_0, r2_1, r2_2, r2_3,
             s0, s1, s2, s3):
    r1 = (r1_0, r1_1, r1_2, r1_3)
    r2 = (r2_0, r2_1, r2_2, r2_3)
    sems = (s0, s1, s2, s3)

    wid = lax.axis_index("s") * NC + lax.axis_index("c")

    pltpu.sync_copy(idx1_hbm.at[wid], idx1_v)
    pltpu.sync_copy(idx2_hbm.at[wid], idx2_v)

    def fire(c, b):
        pltpu.async_copy(emb_hbm.at[idx1_v.at[c]], r1[b], sems[b])
        pltpu.async_copy(emb_hbm.at[idx2_v.at[c]], r2[b], sems[b])

    for b in range(NBUF):
        fire(b, b)

    iota16 = lax.broadcasted_iota(jnp.int32, (16,), 0)
    rot_idx = {k: (iota16 + k) % 16 for k in (8, 4, 2, 1)}
    lane_masks = [iota16 == i2 for i2 in range(16)]

    def body_cg(cg, _):
        for b in range(NBUF):
            c = cg * NBUF + b
            pltpu.make_async_copy(emb_hbm.at[idx1_v.at[c]], r1[b], sems[b]).wait()
            pltpu.make_async_copy(emb_hbm.at[idx2_v.at[c]], r2[b], sems[b]).wait()

            def dot_g(g, _):
                merged = jnp.zeros((16,), jnp.float32)
                for i2 in range(16):
                    i = g * 16 + i2
                    p = (r1[b][i, pl.ds(0, 16)] * r2[b][i, pl.ds(0, 16)]
                         + r1[b][i, pl.ds(16, 16)] * r2[b][i, pl.ds(16, 16)])
                    for k in (8, 4, 2, 1):
                        p = p + jnp.take(p, rot_idx[k])
                    merged = jnp.where(lane_masks[i2], p, merged)
                out_v[pl.ds(c * CHUNK + g * 16, 16)] = merged
                return 0

            lax.fori_loop(0, GRP, dot_g, 0)

            nxt = c + NBUF

            @pl.when(nxt < NCHUNK)
            def _():
                fire(nxt, b)
        return 0

    lax.fori_loop(0, NCHUNK // NBUF, body_cg, 0)

    pltpu.sync_copy(out_v, out_hbm.at[pl.ds(wid * PER_W, PER_W)])


@jax.jit
def kernel(inputs, embeddings):
    idx = (inputs.astype(jnp.int32) * 4).reshape(2, NW, NCHUNK, CHUNK)
    emb_rows = embeddings.shape[0]
    emb_lin = jnp.pad(embeddings, ((0, 0), (0, 128 - D))).reshape(
        4 * emb_rows, D)
    mesh = plsc.VectorSubcoreMesh(core_axis_name="c", subcore_axis_name="s")
    scratch = (
        [pltpu.VMEM((NCHUNK, CHUNK), jnp.int32)] * 2
        + [pltpu.VMEM((PER_W,), jnp.float32)]
        + [pltpu.VMEM((CHUNK, D), jnp.float32)] * (2 * NBUF)
        + [pltpu.SemaphoreType.DMA] * NBUF
    )
    k = functools.partial(
        pl.kernel,
        out_type=jax.ShapeDtypeStruct((N,), jnp.float32),
        mesh=mesh,
        scratch_types=scratch,
        compiler_params=pltpu.CompilerParams(
            needs_layout_passes=False, use_tc_tiling_on_sc=False),
    )(_sc_body)
    out = k(idx[0], idx[1], emb_lin)
    return out.reshape(B, L)
